# baseline (speedup 1.0000x reference)
import functools

import jax
import jax.numpy as jnp
from jax import lax
from jax.experimental import pallas as pl
from jax.experimental.pallas import tpu as pltpu
from jax.experimental.pallas import tpu_sc as plsc

_VOCAB = 100000
_EMB = 32
_HID = 128
_CTX = 3
_BATCH = 1024
_NT = 2048
_GW = 4 * _EMB


def _gather_sc(emb4, idx_flat):
    info = plsc.get_sparse_core_info()
    nc, ns = info.num_cores, info.num_subcores
    nw = nc * ns
    n = idx_flat.shape[0]
    per = n // nw
    mesh = plsc.VectorSubcoreMesh(core_axis_name="c", subcore_axis_name="s")

    @functools.partial(
        pl.kernel,
        mesh=mesh,
        out_type=jax.ShapeDtypeStruct((n, _GW), jnp.float32),
        scratch_types=[
            pltpu.VMEM((per,), jnp.int32),
            pltpu.VMEM((per,), jnp.int32),
            pltpu.VMEM((per, _GW), jnp.float32),
            pltpu.SemaphoreType.DMA,
        ],
    )
    def gather_k(table_hbm, idx_hbm, out_hbm, idx_v, row_v, rows_v, sem):
        wid = lax.axis_index("s") * nc + lax.axis_index("c")
        base = wid * per
        pltpu.sync_copy(idx_hbm.at[pl.ds(base, per)], idx_v)
        for k in range(per // 16):
            sl = pl.ds(k * 16, 16)
            row_v[sl] = lax.shift_right_logical(idx_v[sl], 2)
        pltpu.async_copy(table_hbm.at[row_v], rows_v, sem).wait()
        pltpu.sync_copy(rows_v, out_hbm.at[pl.ds(base, per)])

    return gather_k(emb4, idx_flat)


_NBUF = 4
_GRID = pl.cdiv(_VOCAB, _NT)
_TAIL = _VOCAB - (_GRID - 1) * _NT


def _mlp_body(x_ref, big_ref, w1r_ref, b1_ref, w2_ref, b2_ref, out_hbm,
              hid_ref, obuf, tailbuf, sems, tail_sem):
    s = pl.program_id(0)
    last = _GRID - 1

    @pl.when(s == 0)
    def _():
        xm = x_ref[...] % 4
        xm_b = jnp.concatenate(
            [jnp.broadcast_to(xm[:, c:c + 1], (_BATCH, _GW))
             for c in range(_CTX)], axis=1)
        li = lax.broadcasted_iota(jnp.int32, (_BATCH, _CTX * _GW), 1)
        live = xm_b == (li // _EMB) % 4
        bigm = jnp.where(live, big_ref[...], 0.0)
        h = jnp.dot(bigm, w1r_ref[...], preferred_element_type=jnp.float32)
        h = jnp.maximum(h + b1_ref[...], 0.0)
        hid_ref[...] = h.astype(jnp.bfloat16)

    acc = jnp.dot(hid_ref[...], w2_ref[...].astype(jnp.bfloat16),
                  preferred_element_type=jnp.float32)
    res = acc + b2_ref[...]
    slot = lax.rem(s, _NBUF)

    @pl.when(s >= _NBUF)
    def _():
        prev = s - _NBUF
        pltpu.make_async_copy(
            obuf.at[slot],
            out_hbm.at[:, pl.ds(prev * _NT, _NT)],
            sems.at[slot]).wait()

    @pl.when(s < last)
    def _():
        obuf[slot] = res
        pltpu.make_async_copy(
            obuf.at[slot],
            out_hbm.at[:, pl.ds(s * _NT, _NT)],
            sems.at[slot]).start()

    @pl.when(s == last)
    def _():
        tailbuf[...] = res[:, :_TAIL]
        pltpu.make_async_copy(
            tailbuf,
            out_hbm.at[:, pl.ds(last * _NT, _TAIL)],
            tail_sem).start()
        for k in range(1, _NBUF):
            prev = last - k
            pslot = lax.rem(prev, _NBUF)
            pltpu.make_async_copy(
                obuf.at[pslot],
                out_hbm.at[:, pl.ds(prev * _NT, _NT)],
                sems.at[pslot]).wait()
        pltpu.make_async_copy(
            tailbuf,
            out_hbm.at[:, pl.ds(last * _NT, _TAIL)],
            tail_sem).wait()


def _mlp_tc(x, big, w1r, b1, w2, b2):
    return pl.pallas_call(
        _mlp_body,
        grid=(_GRID,),
        in_specs=[
            pl.BlockSpec((_BATCH, _CTX), lambda i: (0, 0)),
            pl.BlockSpec((_BATCH, _CTX * _GW), lambda i: (0, 0)),
            pl.BlockSpec((_CTX * _GW, _HID), lambda i: (0, 0)),
            pl.BlockSpec((1, _HID), lambda i: (0, 0)),
            pl.BlockSpec((_HID, _NT), lambda i: (0, i)),
            pl.BlockSpec((1, _NT), lambda i: (0, i)),
        ],
        out_specs=pl.BlockSpec(memory_space=pl.ANY),
        out_shape=jax.ShapeDtypeStruct((_BATCH, _VOCAB), jnp.float32),
        scratch_shapes=[
            pltpu.VMEM((_BATCH, _HID), jnp.bfloat16),
            pltpu.VMEM((_NBUF, _BATCH, _NT), jnp.float32),
            pltpu.VMEM((_BATCH, _TAIL), jnp.float32),
            pltpu.SemaphoreType.DMA((_NBUF,)),
            pltpu.SemaphoreType.DMA,
        ],
        compiler_params=pltpu.CompilerParams(
            dimension_semantics=("arbitrary",),
        ),
    )(x, big, w1r, b1, w2, b2)


def kernel(x, emb, W1, b1, W2, b2):
    x = x.astype(jnp.int32)
    idx = x.reshape(-1)
    emb4 = emb.reshape(_VOCAB // 4, _GW)
    big = _gather_sc(emb4, idx).reshape(_BATCH, _CTX * _GW)
    w1r = jnp.broadcast_to(
        W1.reshape(_CTX, 1, _EMB, _HID),
        (_CTX, 4, _EMB, _HID)).reshape(_CTX * _GW, _HID)
    return _mlp_tc(x, big, w1r, b1.reshape(1, -1), W2, b2.reshape(1, -1))


# speedup vs baseline: 2.9744x; 2.9744x over previous
import functools

import jax
import jax.numpy as jnp
from jax import lax
from jax.experimental import pallas as pl
from jax.experimental.pallas import tpu as pltpu
from jax.experimental.pallas import tpu_sc as plsc

_VOCAB = 100000
_EMB = 32
_HID = 128
_CTX = 3
_BATCH = 1024
_NT = 2048
_NE = 8192
_NEH = _NE // 2
_EGRID = (_VOCAB + _NE - 1) // _NE
_HROWS = _EGRID * _NEH


def _ew_body(embt_ref, w1_ref, out_ref):
    v = embt_ref[...].astype(jnp.bfloat16)
    parts = []
    for c in range(_CTX):
        w1c = w1_ref[pl.ds(c * _EMB, _EMB), :].astype(jnp.bfloat16)
        acc = lax.dot_general(v, w1c, (((0,), (0,)), ((), ())),
                              preferred_element_type=jnp.float32)
        u = lax.bitcast_convert_type(acc, jnp.uint32)
        lo = (u[:_NEH, :] + jnp.uint32(0x8000)) >> 16
        hi = u[_NEH:, :] & jnp.uint32(0xFFFF0000)
        parts.append(lax.bitcast_convert_type(lo | hi, jnp.int32)[None])
    out_ref[...] = jnp.concatenate(parts, axis=0)


def _ew_tc(embt, w1):
    return pl.pallas_call(
        _ew_body,
        grid=(_EGRID,),
        in_specs=[
            pl.BlockSpec((_EMB, _NE), lambda i: (0, i)),
            pl.BlockSpec((_CTX * _EMB, _HID), lambda i: (0, 0)),
        ],
        out_specs=pl.BlockSpec((_CTX, _NEH, _HID), lambda i: (0, i, 0)),
        out_shape=jax.ShapeDtypeStruct((_CTX, _HROWS, _HID), jnp.int32),
    )(embt, w1)


def _gather_sc(ews, idx_flat):
    info = plsc.get_sparse_core_info()
    nc, ns = info.num_cores, info.num_subcores
    nw = nc * ns
    n = idx_flat.shape[0]
    per = n // nw
    mesh = plsc.VectorSubcoreMesh(core_axis_name="c", subcore_axis_name="s")

    @functools.partial(
        pl.kernel,
        mesh=mesh,
        out_type=jax.ShapeDtypeStruct((n, _HID), jnp.int32),
        scratch_types=[
            pltpu.VMEM((per,), jnp.int32),
            pltpu.VMEM((per,), jnp.int32),
            pltpu.VMEM((per, _HID), jnp.int32),
            pltpu.SemaphoreType.DMA,
        ],
    )
    def gather_k(tab_hbm, idx_hbm, out_hbm, idx_v, row_v, rows_v, sem):
        wid = lax.axis_index("s") * nc + lax.axis_index("c")
        base = wid * per
        pltpu.sync_copy(idx_hbm.at[pl.ds(base, per)], idx_v)
        for k in range(per // 16):
            sl = pl.ds(k * 16, 16)
            cvec = lax.rem(lax.iota(jnp.int32, 16) + (16 * k), 3)
            xi = idx_v[sl]
            blk = lax.shift_left(lax.shift_right_logical(xi, 13), 12)
            row_v[sl] = blk + (xi & (_NEH - 1)) + cvec * _HROWS
        pltpu.async_copy(tab_hbm.at[row_v], rows_v, sem).wait()
        pltpu.sync_copy(rows_v, out_hbm.at[pl.ds(base, per)])

    return gather_k(ews, idx_flat)


def _mlp_body(x_ref, g_ref, b1_ref, w2t_ref, b2_ref, out_ref, hidt_ref):
    @pl.when(pl.program_id(0) == 0)
    def _():
        u = lax.bitcast_convert_type(g_ref[...], jnp.uint32)
        par = ((x_ref[...] >> 12) & 1)[:, :, None]
        parb = jnp.broadcast_to(par, (_BATCH, _CTX, _HID))
        bits = jnp.where(parb == 1, u & jnp.uint32(0xFFFF0000), u << 16)
        f = lax.bitcast_convert_type(bits, jnp.float32)
        h = f[:, 0, :] + f[:, 1, :] + f[:, 2, :]
        h = jnp.maximum(h + b1_ref[...], 0.0)
        hidt_ref[...] = jnp.transpose(h).astype(jnp.bfloat16)

    acc = jnp.dot(w2t_ref[...].astype(jnp.bfloat16), hidt_ref[...],
                  preferred_element_type=jnp.float32)
    out_ref[...] = acc + jnp.transpose(b2_ref[...])


def _mlp_tc(x, g, b1, w2t, b2):
    grid = pl.cdiv(_VOCAB, _NT)
    return pl.pallas_call(
        _mlp_body,
        grid=(grid,),
        in_specs=[
            pl.BlockSpec((_BATCH, _CTX), lambda i: (0, 0)),
            pl.BlockSpec((_BATCH, _CTX, _HID), lambda i: (0, 0, 0)),
            pl.BlockSpec((1, _HID), lambda i: (0, 0)),
            pl.BlockSpec((_NT, _HID), lambda i: (i, 0)),
            pl.BlockSpec((1, _NT), lambda i: (0, i)),
        ],
        out_specs=pl.BlockSpec((_NT, _BATCH), lambda i: (i, 0)),
        out_shape=jax.ShapeDtypeStruct((_VOCAB, _BATCH), jnp.float32),
        scratch_shapes=[pltpu.VMEM((_HID, _BATCH), jnp.bfloat16)],
        compiler_params=pltpu.CompilerParams(
            dimension_semantics=("arbitrary",),
        ),
    )(x, g, b1, w2t, b2)


def kernel(x, emb, W1, b1, W2, b2):
    x = x.astype(jnp.int32)
    idx = x.reshape(-1)
    ew = _ew_tc(emb.T, W1)
    ews = ew.reshape(_CTX * _HROWS, _HID)
    g = _gather_sc(ews, idx).reshape(_BATCH, _CTX, _HID)
    out_t = _mlp_tc(x, g, b1.reshape(1, -1), W2.T, b2.reshape(1, -1))
    return out_t.T


# speedup vs baseline: 3.1123x; 1.0464x over previous
import functools

import jax
import jax.numpy as jnp
from jax import lax
from jax.experimental import pallas as pl
from jax.experimental.pallas import tpu as pltpu
from jax.experimental.pallas import tpu_sc as plsc

_VOCAB = 100000
_EMB = 32
_HID = 128
_CTX = 3
_BATCH = 1024
_NT = 4096
_NE = 8192
_NEH = _NE // 2
_EGRID = (_VOCAB + _NE - 1) // _NE
_HROWS = _EGRID * _NEH


def _ew_body(embt_ref, w1_ref, out_ref):
    v = embt_ref[...].astype(jnp.bfloat16)
    parts = []
    for c in range(_CTX):
        w1c = w1_ref[pl.ds(c * _EMB, _EMB), :].astype(jnp.bfloat16)
        acc = lax.dot_general(v, w1c, (((0,), (0,)), ((), ())),
                              preferred_element_type=jnp.float32)
        u = lax.bitcast_convert_type(acc, jnp.uint32)
        lo = (u[:_NEH, :] + jnp.uint32(0x8000)) >> 16
        hi = u[_NEH:, :] & jnp.uint32(0xFFFF0000)
        parts.append(lax.bitcast_convert_type(lo | hi, jnp.int32)[None])
    out_ref[...] = jnp.concatenate(parts, axis=0)


def _ew_tc(embt, w1):
    return pl.pallas_call(
        _ew_body,
        grid=(_EGRID,),
        in_specs=[
            pl.BlockSpec((_EMB, _NE), lambda i: (0, i)),
            pl.BlockSpec((_CTX * _EMB, _HID), lambda i: (0, 0)),
        ],
        out_specs=pl.BlockSpec((_CTX, _NEH, _HID), lambda i: (0, i, 0)),
        out_shape=jax.ShapeDtypeStruct((_CTX, _HROWS, _HID), jnp.int32),
    )(embt, w1)


def _gather_sc(ews, idx_flat):
    info = plsc.get_sparse_core_info()
    nc, ns = info.num_cores, info.num_subcores
    nw = nc * ns
    n = idx_flat.shape[0]
    per = n // nw
    mesh = plsc.VectorSubcoreMesh(core_axis_name="c", subcore_axis_name="s")

    @functools.partial(
        pl.kernel,
        mesh=mesh,
        out_type=jax.ShapeDtypeStruct((_CTX * _BATCH, _HID), jnp.int32),
        scratch_types=[
            pltpu.VMEM((per,), jnp.int32),
            pltpu.VMEM((per,), jnp.int32),
            pltpu.VMEM((per, _HID), jnp.int32),
            pltpu.SemaphoreType.DMA,
        ],
    )
    def gather_k(tab_hbm, idx_hbm, out_hbm, idx_v, row_v, rows_v, sem):
        wid = lax.axis_index("s") * nc + lax.axis_index("c")
        base = wid * per
        pltpu.sync_copy(idx_hbm.at[pl.ds(base, per)], idx_v)
        for k in range(per // 16):
            sl = pl.ds(k * 16, 16)
            r = lax.iota(jnp.int32, 16) + (base + 16 * k)
            cvec = lax.shift_right_logical(r, 10)
            xi = idx_v[sl]
            blk = lax.shift_left(lax.shift_right_logical(xi, 13), 12)
            row_v[sl] = blk + (xi & (_NEH - 1)) + cvec * _HROWS
        pltpu.async_copy(tab_hbm.at[row_v], rows_v, sem).wait()
        pltpu.sync_copy(rows_v, out_hbm.at[pl.ds(base, per)])

    return gather_k(ews, idx_flat)


def _mlp_body(x_ref, g_ref, b1_ref, w2t_ref, b2_ref, out_ref, hidt_ref):
    @pl.when(pl.program_id(0) == 0)
    def _():
        h = jnp.zeros((_BATCH, _HID), jnp.float32)
        for c in range(_CTX):
            u = lax.bitcast_convert_type(g_ref[c], jnp.uint32)
            par = ((x_ref[:, c:c + 1] >> 12) & 1)
            parb = jnp.broadcast_to(par, (_BATCH, _HID))
            bits = jnp.where(parb == 1, u & jnp.uint32(0xFFFF0000), u << 16)
            h = h + lax.bitcast_convert_type(bits, jnp.float32)
        h = jnp.maximum(h + b1_ref[...], 0.0)
        hidt_ref[...] = jnp.transpose(h).astype(jnp.bfloat16)

    acc = jnp.dot(w2t_ref[...].astype(jnp.bfloat16), hidt_ref[...],
                  preferred_element_type=jnp.float32)
    out_ref[...] = acc + jnp.transpose(b2_ref[...])


def _mlp_tc(x, g, b1, w2t, b2):
    grid = pl.cdiv(_VOCAB, _NT)
    return pl.pallas_call(
        _mlp_body,
        grid=(grid,),
        in_specs=[
            pl.BlockSpec((_BATCH, _CTX), lambda i: (0, 0)),
            pl.BlockSpec((_CTX, _BATCH, _HID), lambda i: (0, 0, 0)),
            pl.BlockSpec((1, _HID), lambda i: (0, 0)),
            pl.BlockSpec((_NT, _HID), lambda i: (i, 0)),
            pl.BlockSpec((1, _NT), lambda i: (0, i)),
        ],
        out_specs=pl.BlockSpec((_NT, _BATCH), lambda i: (i, 0)),
        out_shape=jax.ShapeDtypeStruct((_VOCAB, _BATCH), jnp.float32),
        scratch_shapes=[pltpu.VMEM((_HID, _BATCH), jnp.bfloat16)],
        compiler_params=pltpu.CompilerParams(
            dimension_semantics=("arbitrary",),
        ),
    )(x, g, b1, w2t, b2)


def kernel(x, emb, W1, b1, W2, b2):
    x = x.astype(jnp.int32)
    idx = x.T.reshape(-1)
    ew = _ew_tc(emb.T, W1)
    ews = ew.reshape(_CTX * _HROWS, _HID)
    g = _gather_sc(ews, idx).reshape(_CTX, _BATCH, _HID)
    out_t = _mlp_tc(x, g, b1.reshape(1, -1), W2.T, b2.reshape(1, -1))
    return out_t.T
